# baseline (speedup 1.0000x reference)
import functools

import jax
import jax.numpy as jnp
from jax import lax
from jax.experimental import pallas as pl
from jax.experimental.pallas import tpu as pltpu
from jax.experimental.pallas import tpu_sc as plsc

NC = 2
NS = 16
NW = NC * NS
LANE = 128
RING = 6
DEPTH = 3


def _round_up(v, m):
    return (v + m - 1) // m * m


def _make_deg_kernel(npad, cpt):
    rpt = npad // NS
    mesh = plsc.VectorSubcoreMesh(
        core_axis_name="c", subcore_axis_name="s",
        num_cores=NC, num_subcores=NS)

    @functools.partial(
        pl.kernel,
        out_type=jax.ShapeDtypeStruct((NC * npad,), jnp.float32),
        mesh=mesh,
        scratch_types=[
            pltpu.VMEM((cpt, LANE), jnp.int32),
            pltpu.VMEM((LANE,), jnp.float32),
            pltpu.VMEM_SHARED((npad,), jnp.float32),
            pltpu.SemaphoreType.DMA,
        ],
    )
    def deg_kernel(dst_hbm, zeros_hbm, ones_hbm, out_hbm, dst_v, ones_v, acc,
                   sem):
        c = lax.axis_index("c")
        s = lax.axis_index("s")
        w = c * NS + s
        pltpu.sync_copy(dst_hbm.at[w], dst_v)
        pltpu.sync_copy(ones_hbm, ones_v)
        pltpu.sync_copy(zeros_hbm.at[pl.ds(s * rpt, rpt)],
                        acc.at[pl.ds(s * rpt, rpt)])
        plsc.subcore_barrier()

        def fire(k, car):
            pltpu.async_copy(ones_v, acc.at[dst_v.at[k]], sem, add=True)
            return car

        lax.fori_loop(0, cpt, fire, 0)

        def drain(k, car):
            pltpu.make_async_copy(ones_v, acc.at[dst_v.at[k]], sem).wait()
            return car

        lax.fori_loop(0, cpt, drain, 0)
        plsc.subcore_barrier()
        pltpu.sync_copy(acc.at[pl.ds(s * rpt, rpt)],
                        out_hbm.at[pl.ds(c * npad + s * rpt, rpt)])

    return deg_kernel


def _make_agg_kernel(npad, cpt, feat):
    rpt = npad // NS
    mesh = plsc.VectorSubcoreMesh(
        core_axis_name="c", subcore_axis_name="s",
        num_cores=NC, num_subcores=NS)

    @functools.partial(
        pl.kernel,
        out_type=jax.ShapeDtypeStruct((2 * NC, npad, feat), jnp.float32),
        mesh=mesh,
        scratch_types=[
            pltpu.VMEM((cpt, LANE), jnp.int32),
            pltpu.VMEM((cpt, LANE), jnp.int32),
            pltpu.VMEM((RING, LANE, feat), jnp.float32),
            pltpu.VMEM_SHARED((2, npad, feat), jnp.float32),
            pltpu.SemaphoreType.DMA((RING,)),
            pltpu.SemaphoreType.DMA((RING,)),
        ],
        compiler_params=pltpu.CompilerParams(use_tc_tiling_on_sc=False),
    )
    def agg_kernel(table_hbm, src_hbm, dst_hbm, zeros_hbm, out_hbm,
                   idx_v, dst_v, rows_v, acc2, gsems, ssems):
        c = lax.axis_index("c")
        s = lax.axis_index("s")
        w = c * NS + s
        par = s % 2
        sub = s // 2
        rpt2 = npad // (NS // 2)
        acc = acc2.at[par]
        pltpu.sync_copy(src_hbm.at[w], idx_v)
        pltpu.sync_copy(dst_hbm.at[w], dst_v)

        for b in range(DEPTH):
            pltpu.async_copy(table_hbm.at[idx_v.at[b]], rows_v.at[b],
                             gsems.at[b])
        pltpu.sync_copy(zeros_hbm.at[pl.ds(sub * rpt2, rpt2)],
                        acc.at[pl.ds(sub * rpt2, rpt2)])
        plsc.subcore_barrier()

        def group(g, car):
            for b in range(RING):
                j = g * RING + b
                bn = (b + DEPTH) % RING
                pltpu.make_async_copy(table_hbm.at[idx_v.at[j]],
                                      rows_v.at[b], gsems.at[b]).wait()
                pltpu.async_copy(rows_v.at[b], acc.at[dst_v.at[j]],
                                 ssems.at[b], add=True)

                @pl.when(j >= DEPTH)
                def _retire():
                    pltpu.make_async_copy(rows_v.at[bn],
                                          acc.at[dst_v.at[j - DEPTH]],
                                          ssems.at[bn]).wait()

                @pl.when(j + DEPTH < cpt)
                def _fire():
                    pltpu.async_copy(table_hbm.at[idx_v.at[j + DEPTH]],
                                     rows_v.at[bn], gsems.at[bn])
            return car

        lax.fori_loop(0, cpt // RING, group, 0)
        for t in range(cpt - DEPTH, cpt):
            pltpu.make_async_copy(rows_v.at[t % RING],
                                  acc.at[dst_v.at[t]],
                                  ssems.at[t % RING]).wait()
        plsc.subcore_barrier()
        pltpu.sync_copy(acc.at[pl.ds(sub * rpt2, rpt2)],
                        out_hbm.at[c * 2 + par, pl.ds(sub * rpt2, rpt2)])

    return agg_kernel


def _tc1_body(x_ref, w_ref, d0_ref, d1_ref, hn_ref, dinv_ref):
    deg = d0_ref[...] + d1_ref[...]
    dinv = lax.rsqrt(deg)
    h = jnp.dot(x_ref[...], w_ref[...], preferred_element_type=jnp.float32)
    hn_ref[...] = h * dinv
    dinv_ref[...] = dinv


def _tc2_body(p0_ref, p1_ref, p2_ref, p3_ref, b_ref, dinv_ref, out_ref):
    dinv = dinv_ref[...]
    agg = ((p0_ref[...] + p1_ref[...]) + (p2_ref[...] + p3_ref[...])) * dinv \
        + b_ref[...]
    out_ref[...] = jnp.maximum(agg, 0.0) * dinv


def _tc3_body(q0_ref, q1_ref, q2_ref, q3_ref, w_ref, b_ref, dinv_ref,
              out_ref):
    sagg = (q0_ref[...] + q1_ref[...]) + (q2_ref[...] + q3_ref[...])
    t = (jnp.dot(sagg, w_ref[...], preferred_element_type=jnp.float32)
         * dinv_ref[...] + b_ref[...])
    m = jnp.max(t, axis=1, keepdims=True)
    lse = jnp.log(jnp.sum(jnp.exp(t - m), axis=1, keepdims=True))
    out_ref[...] = t - m - lse


def kernel(x, edge_index, W1, b1, W2, b2):
    n, d_in = x.shape
    h_dim = W1.shape[1]
    c_dim = W2.shape[1]
    e = edge_index.shape[1]
    et = e + n

    cpt = _round_up(-(-et // (NW * LANE)), RING)
    epad = NW * cpt * LANE
    rpt = _round_up(-(-(n + 1) // NS), 16)
    npad = rpt * NS

    f32 = jnp.float32
    i32 = jnp.int32
    loops = jnp.arange(n, dtype=i32)
    pad_dst = n + jnp.arange(epad - et, dtype=i32) % (npad - n)
    src = jnp.concatenate([edge_index[0], loops,
                           jnp.zeros((epad - et,), i32)]).reshape(NW, cpt, LANE)
    dst = jnp.concatenate([edge_index[1], loops,
                           pad_dst]).reshape(NW, cpt, LANE)
    zeros1 = jnp.zeros((npad,), f32)
    zeros2 = jnp.zeros((npad, h_dim), f32)
    ones = jnp.ones((LANE,), f32)

    deg_parts = _make_deg_kernel(npad, cpt)(dst, zeros1, ones).reshape(NC, npad)
    agg = _make_agg_kernel(npad, cpt, h_dim)

    blk = 1000
    grid = (n // blk,)

    hn1, dinv = pl.pallas_call(
        _tc1_body,
        grid=grid,
        in_specs=[
            pl.BlockSpec((blk, d_in), lambda i: (i, 0)),
            pl.BlockSpec((d_in, h_dim), lambda i: (0, 0)),
            pl.BlockSpec((blk, 1), lambda i: (i, 0)),
            pl.BlockSpec((blk, 1), lambda i: (i, 0)),
        ],
        out_specs=[
            pl.BlockSpec((blk, h_dim), lambda i: (i, 0)),
            pl.BlockSpec((blk, 1), lambda i: (i, 0)),
        ],
        out_shape=[
            jax.ShapeDtypeStruct((n, h_dim), f32),
            jax.ShapeDtypeStruct((n, 1), f32),
        ],
    )(x, W1, deg_parts[0, :n, None], deg_parts[1, :n, None])

    p = agg(hn1, src, dst, zeros2)

    hn2 = pl.pallas_call(
        _tc2_body,
        grid=grid,
        in_specs=[
            pl.BlockSpec((blk, h_dim), lambda i: (i, 0)),
            pl.BlockSpec((blk, h_dim), lambda i: (i, 0)),
            pl.BlockSpec((blk, h_dim), lambda i: (i, 0)),
            pl.BlockSpec((blk, h_dim), lambda i: (i, 0)),
            pl.BlockSpec((1, h_dim), lambda i: (0, 0)),
            pl.BlockSpec((blk, 1), lambda i: (i, 0)),
        ],
        out_specs=pl.BlockSpec((blk, h_dim), lambda i: (i, 0)),
        out_shape=jax.ShapeDtypeStruct((n, h_dim), f32),
    )(p[0, :n], p[1, :n], p[2, :n], p[3, :n], b1[None, :], dinv)

    q = agg(hn2, src, dst, zeros2)

    out = pl.pallas_call(
        _tc3_body,
        grid=grid,
        in_specs=[
            pl.BlockSpec((blk, h_dim), lambda i: (i, 0)),
            pl.BlockSpec((blk, h_dim), lambda i: (i, 0)),
            pl.BlockSpec((blk, h_dim), lambda i: (i, 0)),
            pl.BlockSpec((blk, h_dim), lambda i: (i, 0)),
            pl.BlockSpec((h_dim, c_dim), lambda i: (0, 0)),
            pl.BlockSpec((1, c_dim), lambda i: (0, 0)),
            pl.BlockSpec((blk, 1), lambda i: (i, 0)),
        ],
        out_specs=pl.BlockSpec((blk, c_dim), lambda i: (i, 0)),
        out_shape=jax.ShapeDtypeStruct((n, c_dim), f32),
    )(q[0, :n], q[1, :n], q[2, :n], q[3, :n], W2, b2[None, :], dinv)

    return out


# speedup vs baseline: 2.1795x; 2.1795x over previous
import functools

import jax
import jax.numpy as jnp
from jax import lax
from jax.experimental import pallas as pl
from jax.experimental.pallas import tpu as pltpu
from jax.experimental.pallas import tpu_sc as plsc

NC = 2
NS = 16
NW = NC * NS
LANE = 128
RING = 6
DEPTH = 3


def _round_up(v, m):
    return (v + m - 1) // m * m


def _make_deg_kernel(npad, cpt):
    rpt = npad // NS
    mesh = plsc.VectorSubcoreMesh(
        core_axis_name="c", subcore_axis_name="s",
        num_cores=NC, num_subcores=NS)

    @functools.partial(
        pl.kernel,
        out_type=jax.ShapeDtypeStruct((NC * npad,), jnp.float32),
        mesh=mesh,
        scratch_types=[
            pltpu.VMEM((cpt, LANE), jnp.int32),
            pltpu.VMEM((LANE,), jnp.float32),
            pltpu.VMEM_SHARED((npad,), jnp.float32),
            pltpu.SemaphoreType.DMA,
        ],
    )
    def deg_kernel(dst_hbm, zeros_hbm, ones_hbm, out_hbm, dst_v, ones_v, acc,
                   sem):
        c = lax.axis_index("c")
        s = lax.axis_index("s")
        w = c * NS + s
        pltpu.sync_copy(dst_hbm.at[w], dst_v)
        pltpu.sync_copy(ones_hbm, ones_v)
        pltpu.sync_copy(zeros_hbm.at[pl.ds(s * rpt, rpt)],
                        acc.at[pl.ds(s * rpt, rpt)])
        plsc.subcore_barrier()

        def fire(k, car):
            pltpu.async_copy(ones_v, acc.at[dst_v.at[k]], sem, add=True)
            return car

        lax.fori_loop(0, cpt, fire, 0)

        def drain(k, car):
            pltpu.make_async_copy(ones_v, acc.at[dst_v.at[k]], sem).wait()
            return car

        lax.fori_loop(0, cpt, drain, 0)
        plsc.subcore_barrier()
        pltpu.sync_copy(acc.at[pl.ds(s * rpt, rpt)],
                        out_hbm.at[pl.ds(c * npad + s * rpt, rpt)])

    return deg_kernel


def _make_agg_kernel(n, npad, cpt, feat):
    rpt = npad // NS
    tpt = n // NS
    mesh = plsc.VectorSubcoreMesh(
        core_axis_name="c", subcore_axis_name="s",
        num_cores=NC, num_subcores=NS)

    @functools.partial(
        pl.kernel,
        out_type=jax.ShapeDtypeStruct((NC, npad, feat), jnp.float32),
        mesh=mesh,
        scratch_types=[
            pltpu.VMEM((cpt, LANE), jnp.int32),
            pltpu.VMEM((cpt, LANE), jnp.int32),
            pltpu.VMEM((RING, LANE, feat), jnp.float32),
            pltpu.VMEM_SHARED((npad, feat), jnp.float32),
            pltpu.VMEM_SHARED((n, feat), jnp.float32),
            pltpu.SemaphoreType.DMA((RING,)),
            pltpu.SemaphoreType.DMA((RING,)),
        ],
        compiler_params=pltpu.CompilerParams(use_tc_tiling_on_sc=False),
    )
    def agg_kernel(table_hbm, src_hbm, dst_hbm, zeros_hbm, out_hbm,
                   idx_v, dst_v, rows_v, acc, tab, gsems, ssems):
        c = lax.axis_index("c")
        s = lax.axis_index("s")
        w = c * NS + s
        pltpu.sync_copy(src_hbm.at[w], idx_v)
        pltpu.sync_copy(dst_hbm.at[w], dst_v)

        pltpu.sync_copy(table_hbm.at[pl.ds(s * tpt, tpt)],
                        tab.at[pl.ds(s * tpt, tpt)])
        pltpu.sync_copy(zeros_hbm.at[pl.ds(s * rpt, rpt)],
                        acc.at[pl.ds(s * rpt, rpt)])
        plsc.subcore_barrier()
        for b in range(DEPTH):
            pltpu.async_copy(tab.at[idx_v.at[b]], rows_v.at[b],
                             gsems.at[b])

        def group(g, car):
            for b in range(RING):
                j = g * RING + b
                bn = (b + DEPTH) % RING
                pltpu.make_async_copy(tab.at[idx_v.at[j]],
                                      rows_v.at[b], gsems.at[b]).wait()
                pltpu.async_copy(rows_v.at[b], acc.at[dst_v.at[j]],
                                 ssems.at[b], add=True)

                @pl.when(j >= DEPTH)
                def _retire():
                    pltpu.make_async_copy(rows_v.at[bn],
                                          acc.at[dst_v.at[j - DEPTH]],
                                          ssems.at[bn]).wait()

                @pl.when(j + DEPTH < cpt)
                def _fire():
                    pltpu.async_copy(tab.at[idx_v.at[j + DEPTH]],
                                     rows_v.at[bn], gsems.at[bn])
            return car

        lax.fori_loop(0, cpt // RING, group, 0)
        for t in range(cpt - DEPTH, cpt):
            pltpu.make_async_copy(rows_v.at[t % RING],
                                  acc.at[dst_v.at[t]],
                                  ssems.at[t % RING]).wait()
        plsc.subcore_barrier()
        pltpu.sync_copy(acc.at[pl.ds(s * rpt, rpt)],
                        out_hbm.at[c, pl.ds(s * rpt, rpt)])

    return agg_kernel


def _tc1_body(x_ref, w_ref, d0_ref, d1_ref, hn_ref, dinv_ref):
    deg = d0_ref[...] + d1_ref[...]
    dinv = lax.rsqrt(deg)
    h = jnp.dot(x_ref[...], w_ref[...], preferred_element_type=jnp.float32)
    hn_ref[...] = h * dinv
    dinv_ref[...] = dinv


def _tc2_body(p0_ref, p1_ref, b_ref, dinv_ref, out_ref):
    dinv = dinv_ref[...]
    agg = (p0_ref[...] + p1_ref[...]) * dinv + b_ref[...]
    out_ref[...] = jnp.maximum(agg, 0.0) * dinv


def _tc3_body(q0_ref, q1_ref, w_ref, b_ref, dinv_ref, out_ref):
    sagg = q0_ref[...] + q1_ref[...]
    t = (jnp.dot(sagg, w_ref[...], preferred_element_type=jnp.float32)
         * dinv_ref[...] + b_ref[...])
    m = jnp.max(t, axis=1, keepdims=True)
    lse = jnp.log(jnp.sum(jnp.exp(t - m), axis=1, keepdims=True))
    out_ref[...] = t - m - lse


def kernel(x, edge_index, W1, b1, W2, b2):
    n, d_in = x.shape
    h_dim = W1.shape[1]
    c_dim = W2.shape[1]
    e = edge_index.shape[1]
    et = e + n

    cpt = _round_up(-(-et // (NW * LANE)), RING)
    epad = NW * cpt * LANE
    rpt = _round_up(-(-(n + 1) // NS), 16)
    npad = rpt * NS

    f32 = jnp.float32
    i32 = jnp.int32
    loops = jnp.arange(n, dtype=i32)
    pad_dst = n + jnp.arange(epad - et, dtype=i32) % (npad - n)
    src = jnp.concatenate([edge_index[0], loops,
                           jnp.zeros((epad - et,), i32)]).reshape(NW, cpt, LANE)
    dst = jnp.concatenate([edge_index[1], loops,
                           pad_dst]).reshape(NW, cpt, LANE)
    zeros1 = jnp.zeros((npad,), f32)
    zeros2 = jnp.zeros((npad, h_dim), f32)
    ones = jnp.ones((LANE,), f32)

    deg_parts = _make_deg_kernel(npad, cpt)(dst, zeros1, ones).reshape(NC, npad)
    agg = _make_agg_kernel(n, npad, cpt, h_dim)

    blk = 1000
    grid = (n // blk,)

    hn1, dinv = pl.pallas_call(
        _tc1_body,
        grid=grid,
        in_specs=[
            pl.BlockSpec((blk, d_in), lambda i: (i, 0)),
            pl.BlockSpec((d_in, h_dim), lambda i: (0, 0)),
            pl.BlockSpec((blk, 1), lambda i: (i, 0)),
            pl.BlockSpec((blk, 1), lambda i: (i, 0)),
        ],
        out_specs=[
            pl.BlockSpec((blk, h_dim), lambda i: (i, 0)),
            pl.BlockSpec((blk, 1), lambda i: (i, 0)),
        ],
        out_shape=[
            jax.ShapeDtypeStruct((n, h_dim), f32),
            jax.ShapeDtypeStruct((n, 1), f32),
        ],
    )(x, W1, deg_parts[0, :n, None], deg_parts[1, :n, None])

    p = agg(hn1, src, dst, zeros2)

    hn2 = pl.pallas_call(
        _tc2_body,
        grid=grid,
        in_specs=[
            pl.BlockSpec((blk, h_dim), lambda i: (i, 0)),
            pl.BlockSpec((blk, h_dim), lambda i: (i, 0)),
            pl.BlockSpec((1, h_dim), lambda i: (0, 0)),
            pl.BlockSpec((blk, 1), lambda i: (i, 0)),
        ],
        out_specs=pl.BlockSpec((blk, h_dim), lambda i: (i, 0)),
        out_shape=jax.ShapeDtypeStruct((n, h_dim), f32),
    )(p[0, :n], p[1, :n], b1[None, :], dinv)

    q = agg(hn2, src, dst, zeros2)

    out = pl.pallas_call(
        _tc3_body,
        grid=grid,
        in_specs=[
            pl.BlockSpec((blk, h_dim), lambda i: (i, 0)),
            pl.BlockSpec((blk, h_dim), lambda i: (i, 0)),
            pl.BlockSpec((h_dim, c_dim), lambda i: (0, 0)),
            pl.BlockSpec((1, c_dim), lambda i: (0, 0)),
            pl.BlockSpec((blk, 1), lambda i: (i, 0)),
        ],
        out_specs=pl.BlockSpec((blk, c_dim), lambda i: (i, 0)),
        out_shape=jax.ShapeDtypeStruct((n, c_dim), f32),
    )(q[0, :n], q[1, :n], W2, b2[None, :], dinv)

    return out


# speedup vs baseline: 2.2490x; 1.0319x over previous
import functools

import jax
import jax.numpy as jnp
from jax import lax
from jax.experimental import pallas as pl
from jax.experimental.pallas import tpu as pltpu
from jax.experimental.pallas import tpu_sc as plsc

NC = 2
NS = 16
NW = NC * NS
LANE = 128
RING = 6
DEPTH = 3


def _round_up(v, m):
    return (v + m - 1) // m * m


def _make_deg_kernel(npad, cpt):
    rpt = npad // NS
    mesh = plsc.VectorSubcoreMesh(
        core_axis_name="c", subcore_axis_name="s",
        num_cores=NC, num_subcores=NS)

    @functools.partial(
        pl.kernel,
        out_type=jax.ShapeDtypeStruct((NC * npad,), jnp.float32),
        mesh=mesh,
        scratch_types=[
            pltpu.VMEM((cpt, LANE), jnp.int32),
            pltpu.VMEM((LANE,), jnp.float32),
            pltpu.VMEM_SHARED((npad,), jnp.float32),
            pltpu.SemaphoreType.DMA,
        ],
    )
    def deg_kernel(dst_hbm, zeros_hbm, ones_hbm, out_hbm, dst_v, ones_v, acc,
                   sem):
        c = lax.axis_index("c")
        s = lax.axis_index("s")
        w = c * NS + s
        pltpu.sync_copy(dst_hbm.at[w], dst_v)
        pltpu.sync_copy(ones_hbm, ones_v)
        pltpu.sync_copy(zeros_hbm.at[pl.ds(s * rpt, rpt)],
                        acc.at[pl.ds(s * rpt, rpt)])
        plsc.subcore_barrier()

        def fire(k, car):
            pltpu.async_copy(ones_v, acc.at[dst_v.at[k]], sem, add=True)
            return car

        lax.fori_loop(0, cpt, fire, 0)

        def drain(k, car):
            pltpu.make_async_copy(ones_v, acc.at[dst_v.at[k]], sem).wait()
            return car

        lax.fori_loop(0, cpt, drain, 0)
        plsc.subcore_barrier()
        pltpu.sync_copy(acc.at[pl.ds(s * rpt, rpt)],
                        out_hbm.at[pl.ds(c * npad + s * rpt, rpt)])

    return deg_kernel


def _make_agg_kernel(n, npad, cpt, feat):
    rpt = npad // NS
    tpt = n // NS
    mesh = plsc.VectorSubcoreMesh(
        core_axis_name="c", subcore_axis_name="s",
        num_cores=NC, num_subcores=NS)

    @functools.partial(
        pl.kernel,
        out_type=jax.ShapeDtypeStruct((NC, npad, feat), jnp.float32),
        mesh=mesh,
        scratch_types=[
            pltpu.VMEM((cpt, LANE), jnp.int32),
            pltpu.VMEM((cpt, LANE), jnp.int32),
            pltpu.VMEM((RING, LANE, feat), jnp.float32),
            pltpu.VMEM_SHARED((npad, feat), jnp.float32),
            pltpu.VMEM_SHARED((n, feat), jnp.float32),
            pltpu.SemaphoreType.DMA((RING,)),
            pltpu.SemaphoreType.DMA((RING,)),
        ],
        compiler_params=pltpu.CompilerParams(use_tc_tiling_on_sc=False),
    )
    def agg_kernel(table_hbm, src_hbm, dst_hbm, zeros_hbm, out_hbm,
                   idx_v, dst_v, rows_v, acc, tab, gsems, ssems):
        c = lax.axis_index("c")
        s = lax.axis_index("s")
        w = c * NS + s
        pltpu.sync_copy(src_hbm.at[w], idx_v)
        pltpu.sync_copy(dst_hbm.at[w], dst_v)

        pltpu.sync_copy(table_hbm.at[pl.ds(s * tpt, tpt)],
                        tab.at[pl.ds(s * tpt, tpt)])
        pltpu.sync_copy(zeros_hbm.at[pl.ds(s * rpt, rpt)],
                        acc.at[pl.ds(s * rpt, rpt)])
        plsc.subcore_barrier()
        for b in range(DEPTH):
            pltpu.async_copy(tab.at[idx_v.at[b]], rows_v.at[b],
                             gsems.at[b])

        def group(g, car):
            for b in range(RING):
                j = g * RING + b
                bn = (b + DEPTH) % RING
                pltpu.make_async_copy(tab.at[idx_v.at[j]],
                                      rows_v.at[b], gsems.at[b]).wait()
                pltpu.async_copy(rows_v.at[b], acc.at[dst_v.at[j]],
                                 ssems.at[b], add=True)

                @pl.when(j >= DEPTH)
                def _retire():
                    pltpu.make_async_copy(rows_v.at[bn],
                                          acc.at[dst_v.at[j - DEPTH]],
                                          ssems.at[bn]).wait()

                @pl.when(j + DEPTH < cpt)
                def _fire():
                    pltpu.async_copy(tab.at[idx_v.at[j + DEPTH]],
                                     rows_v.at[bn], gsems.at[bn])
            return car

        lax.fori_loop(0, cpt // RING, group, 0)
        for t in range(cpt - DEPTH, cpt):
            pltpu.make_async_copy(rows_v.at[t % RING],
                                  acc.at[dst_v.at[t]],
                                  ssems.at[t % RING]).wait()
        plsc.subcore_barrier()
        pltpu.sync_copy(acc.at[pl.ds(s * rpt, rpt)],
                        out_hbm.at[c, pl.ds(s * rpt, rpt)])

    return agg_kernel


def _tc1_body(x_ref, w_ref, d0_ref, d1_ref, hn_ref, dinv_ref):
    deg = d0_ref[...] + d1_ref[...]
    dinv = lax.rsqrt(deg)
    h = jnp.dot(x_ref[...], w_ref[...], preferred_element_type=jnp.float32)
    hn_ref[...] = h * dinv
    dinv_ref[...] = dinv


def _tc2_body(p0_ref, p1_ref, b_ref, dinv_ref, out_ref):
    dinv = dinv_ref[...]
    agg = (p0_ref[...] + p1_ref[...]) * dinv + b_ref[...]
    out_ref[...] = jnp.maximum(agg, 0.0) * dinv


def _tc3_body(q0_ref, q1_ref, w_ref, b_ref, dinv_ref, out_ref):
    sagg = q0_ref[...] + q1_ref[...]
    t = (jnp.dot(sagg, w_ref[...], preferred_element_type=jnp.float32)
         * dinv_ref[...] + b_ref[...])
    m = jnp.max(t, axis=1, keepdims=True)
    lse = jnp.log(jnp.sum(jnp.exp(t - m), axis=1, keepdims=True))
    out_ref[...] = t - m - lse


def kernel(x, edge_index, W1, b1, W2, b2):
    n, d_in = x.shape
    h_dim = W1.shape[1]
    c_dim = W2.shape[1]
    e = edge_index.shape[1]
    et = e + n

    cpt = _round_up(-(-et // (NW * LANE)), RING)
    epad = NW * cpt * LANE
    rpt = _round_up(-(-(n + 1) // NS), 16)
    npad = rpt * NS

    f32 = jnp.float32
    i32 = jnp.int32
    loops = jnp.arange(n, dtype=i32)
    pad_dst = n + jnp.arange(epad - et, dtype=i32) % (npad - n)
    src = jnp.concatenate([edge_index[0], loops,
                           jnp.zeros((epad - et,), i32)]).reshape(NW, cpt, LANE)
    dst = jnp.concatenate([edge_index[1], loops,
                           pad_dst]).reshape(NW, cpt, LANE)
    zeros1 = jnp.zeros((npad,), f32)
    zeros2 = jnp.zeros((npad, h_dim), f32)
    ones = jnp.ones((LANE,), f32)

    deg_parts = _make_deg_kernel(npad, cpt)(dst, zeros1, ones).reshape(NC, npad)
    agg = _make_agg_kernel(n, npad, cpt, h_dim)

    blk = n
    grid = (1,)

    hn1, dinv = pl.pallas_call(
        _tc1_body,
        grid=grid,
        in_specs=[
            pl.BlockSpec((blk, d_in), lambda i: (i, 0)),
            pl.BlockSpec((d_in, h_dim), lambda i: (0, 0)),
            pl.BlockSpec((blk, 1), lambda i: (i, 0)),
            pl.BlockSpec((blk, 1), lambda i: (i, 0)),
        ],
        out_specs=[
            pl.BlockSpec((blk, h_dim), lambda i: (i, 0)),
            pl.BlockSpec((blk, 1), lambda i: (i, 0)),
        ],
        out_shape=[
            jax.ShapeDtypeStruct((n, h_dim), f32),
            jax.ShapeDtypeStruct((n, 1), f32),
        ],
    )(x, W1, deg_parts[0, :n, None], deg_parts[1, :n, None])

    p = agg(hn1, src, dst, zeros2)

    hn2 = pl.pallas_call(
        _tc2_body,
        grid=grid,
        in_specs=[
            pl.BlockSpec((blk, h_dim), lambda i: (i, 0)),
            pl.BlockSpec((blk, h_dim), lambda i: (i, 0)),
            pl.BlockSpec((1, h_dim), lambda i: (0, 0)),
            pl.BlockSpec((blk, 1), lambda i: (i, 0)),
        ],
        out_specs=pl.BlockSpec((blk, h_dim), lambda i: (i, 0)),
        out_shape=jax.ShapeDtypeStruct((n, h_dim), f32),
    )(p[0, :n], p[1, :n], b1[None, :], dinv)

    q = agg(hn2, src, dst, zeros2)

    out = pl.pallas_call(
        _tc3_body,
        grid=grid,
        in_specs=[
            pl.BlockSpec((blk, h_dim), lambda i: (i, 0)),
            pl.BlockSpec((blk, h_dim), lambda i: (i, 0)),
            pl.BlockSpec((h_dim, c_dim), lambda i: (0, 0)),
            pl.BlockSpec((1, c_dim), lambda i: (0, 0)),
            pl.BlockSpec((blk, 1), lambda i: (i, 0)),
        ],
        out_specs=pl.BlockSpec((blk, c_dim), lambda i: (i, 0)),
        out_shape=jax.ShapeDtypeStruct((n, c_dim), f32),
    )(q[0, :n], q[1, :n], W2, b2[None, :], dinv)

    return out


# speedup vs baseline: 2.5334x; 1.1264x over previous
import functools

import jax
import jax.numpy as jnp
from jax import lax
from jax.experimental import pallas as pl
from jax.experimental.pallas import tpu as pltpu
from jax.experimental.pallas import tpu_sc as plsc

NC = 2
NS = 16
NW = NC * NS
LANE = 128
RING = 6
DEPTH = 3


def _round_up(v, m):
    return (v + m - 1) // m * m


def _make_deg_kernel(npad, cpt):
    rpt = npad // NS
    mesh = plsc.VectorSubcoreMesh(
        core_axis_name="c", subcore_axis_name="s",
        num_cores=NC, num_subcores=NS)

    @functools.partial(
        pl.kernel,
        out_type=jax.ShapeDtypeStruct((NC * npad,), jnp.float32),
        mesh=mesh,
        scratch_types=[
            pltpu.VMEM((cpt, LANE), jnp.int32),
            pltpu.VMEM((LANE,), jnp.float32),
            pltpu.VMEM_SHARED((npad,), jnp.float32),
            pltpu.SemaphoreType.DMA,
        ],
    )
    def deg_kernel(dst_hbm, zeros_hbm, ones_hbm, out_hbm, dst_v, ones_v, acc,
                   sem):
        c = lax.axis_index("c")
        s = lax.axis_index("s")
        w = c * NS + s
        pltpu.sync_copy(dst_hbm.at[w], dst_v)
        pltpu.sync_copy(ones_hbm, ones_v)
        pltpu.sync_copy(zeros_hbm.at[pl.ds(s * rpt, rpt)],
                        acc.at[pl.ds(s * rpt, rpt)])
        plsc.subcore_barrier()

        def fire(k, car):
            pltpu.async_copy(ones_v, acc.at[dst_v.at[k]], sem, add=True)
            return car

        lax.fori_loop(0, cpt, fire, 0)

        def drain(k, car):
            pltpu.make_async_copy(ones_v, acc.at[dst_v.at[k]], sem).wait()
            return car

        lax.fori_loop(0, cpt, drain, 0)
        plsc.subcore_barrier()
        pltpu.sync_copy(acc.at[pl.ds(s * rpt, rpt)],
                        out_hbm.at[pl.ds(c * npad + s * rpt, rpt)])

    return deg_kernel


def _make_agg_kernel(n, npad, cpt, feat, fuse_stage=False):
    rpt = npad // NS
    tpt = n // NS
    tab_rows = npad if fuse_stage else n
    mesh = plsc.VectorSubcoreMesh(
        core_axis_name="c", subcore_axis_name="s",
        num_cores=NC, num_subcores=NS)

    scratch = [
        pltpu.VMEM((cpt, LANE), jnp.int32),
        pltpu.VMEM((cpt, LANE), jnp.int32),
        pltpu.VMEM((RING, LANE, feat), jnp.float32),
        pltpu.VMEM_SHARED((npad, feat), jnp.float32),
        pltpu.VMEM_SHARED((tab_rows, feat), jnp.float32),
        pltpu.SemaphoreType.DMA((RING,)),
        pltpu.SemaphoreType.DMA((RING,)),
    ]
    if fuse_stage:
        scratch += [
            pltpu.VMEM((rpt, feat), jnp.float32),
            pltpu.VMEM((rpt, feat), jnp.float32),
            pltpu.VMEM((rpt, feat), jnp.float32),
            pltpu.VMEM((rpt, feat), jnp.float32),
            pltpu.VMEM((16,), jnp.float32),
        ]

    def _edge_pipeline(s, c, idx_v, dst_v, rows_v, acc, tab, gsems, ssems,
                       out_hbm):
        for b in range(DEPTH):
            pltpu.async_copy(tab.at[idx_v.at[b]], rows_v.at[b],
                             gsems.at[b])

        def group(g, car):
            for b in range(RING):
                j = g * RING + b
                bn = (b + DEPTH) % RING
                pltpu.make_async_copy(tab.at[idx_v.at[j]],
                                      rows_v.at[b], gsems.at[b]).wait()
                pltpu.async_copy(rows_v.at[b], acc.at[dst_v.at[j]],
                                 ssems.at[b], add=True)

                @pl.when(j >= DEPTH)
                def _retire():
                    pltpu.make_async_copy(rows_v.at[bn],
                                          acc.at[dst_v.at[j - DEPTH]],
                                          ssems.at[bn]).wait()

                @pl.when(j + DEPTH < cpt)
                def _fire():
                    pltpu.async_copy(tab.at[idx_v.at[j + DEPTH]],
                                     rows_v.at[bn], gsems.at[bn])
            return car

        lax.fori_loop(0, cpt // RING, group, 0)
        for t in range(cpt - DEPTH, cpt):
            pltpu.make_async_copy(rows_v.at[t % RING],
                                  acc.at[dst_v.at[t]],
                                  ssems.at[t % RING]).wait()
        plsc.subcore_barrier()
        pltpu.sync_copy(acc.at[pl.ds(s * rpt, rpt)],
                        out_hbm.at[c, pl.ds(s * rpt, rpt)])

    if not fuse_stage:
        @functools.partial(
            pl.kernel,
            out_type=jax.ShapeDtypeStruct((NC, npad, feat), jnp.float32),
            mesh=mesh, scratch_types=scratch,
            compiler_params=pltpu.CompilerParams(use_tc_tiling_on_sc=False),
        )
        def agg_kernel(table_hbm, src_hbm, dst_hbm, zeros_hbm, out_hbm,
                       idx_v, dst_v, rows_v, acc, tab, gsems, ssems):
            c = lax.axis_index("c")
            s = lax.axis_index("s")
            w = c * NS + s
            pltpu.sync_copy(src_hbm.at[w], idx_v)
            pltpu.sync_copy(dst_hbm.at[w], dst_v)
            pltpu.sync_copy(table_hbm.at[pl.ds(s * tpt, tpt)],
                            tab.at[pl.ds(s * tpt, tpt)])
            pltpu.sync_copy(zeros_hbm.at[pl.ds(s * rpt, rpt)],
                            acc.at[pl.ds(s * rpt, rpt)])
            plsc.subcore_barrier()
            _edge_pipeline(s, c, idx_v, dst_v, rows_v, acc, tab,
                           gsems, ssems, out_hbm)
        return agg_kernel

    @functools.partial(
        pl.kernel,
        out_type=jax.ShapeDtypeStruct((NC, npad, feat), jnp.float32),
        mesh=mesh, scratch_types=scratch,
        compiler_params=pltpu.CompilerParams(use_tc_tiling_on_sc=False),
    )
    def agg_fused_kernel(p_hbm, dinv_hbm, bias_hbm, src_hbm, dst_hbm,
                         zeros_hbm, out_hbm,
                         idx_v, dst_v, rows_v, acc, tab, gsems, ssems,
                         pa_v, pb_v, st_v, dv_v, b_v):
        c = lax.axis_index("c")
        s = lax.axis_index("s")
        w = c * NS + s
        r0 = s * rpt
        pltpu.sync_copy(src_hbm.at[w], idx_v)
        pltpu.sync_copy(dst_hbm.at[w], dst_v)
        pltpu.sync_copy(p_hbm.at[0, pl.ds(r0, rpt)], pa_v)
        pltpu.sync_copy(p_hbm.at[1, pl.ds(r0, rpt)], pb_v)
        pltpu.sync_copy(dinv_hbm.at[pl.ds(r0, rpt)], dv_v)
        pltpu.sync_copy(bias_hbm, b_v)
        pltpu.sync_copy(zeros_hbm.at[pl.ds(r0, rpt)],
                        acc.at[pl.ds(r0, rpt)])
        bias = b_v[...]

        def rowbody(r, car):
            d = dv_v[r]
            row = (pa_v[r] + pb_v[r]) * d + bias
            st_v[r] = jnp.maximum(row, 0.0) * d
            return car

        lax.fori_loop(0, rpt, rowbody, 0)
        pltpu.sync_copy(st_v, tab.at[pl.ds(r0, rpt)])
        plsc.subcore_barrier()
        _edge_pipeline(s, c, idx_v, dst_v, rows_v, acc, tab,
                       gsems, ssems, out_hbm)

    return agg_fused_kernel


def _tc1_body(x_ref, w_ref, d0_ref, d1_ref, hn_ref, dinv_ref):
    deg = d0_ref[...] + d1_ref[...]
    dinv = lax.rsqrt(deg)
    h = jnp.dot(x_ref[...], w_ref[...], preferred_element_type=jnp.float32)
    hn_ref[...] = h * dinv
    dinv_ref[...] = dinv


def _tc2_body(p0_ref, p1_ref, b_ref, dinv_ref, out_ref):
    dinv = dinv_ref[...]
    agg = (p0_ref[...] + p1_ref[...]) * dinv + b_ref[...]
    out_ref[...] = jnp.maximum(agg, 0.0) * dinv


def _tc3_body(q0_ref, q1_ref, w_ref, b_ref, dinv_ref, out_ref):
    sagg = q0_ref[...] + q1_ref[...]
    t = (jnp.dot(sagg, w_ref[...], preferred_element_type=jnp.float32)
         * dinv_ref[...] + b_ref[...])
    m = jnp.max(t, axis=1, keepdims=True)
    lse = jnp.log(jnp.sum(jnp.exp(t - m), axis=1, keepdims=True))
    out_ref[...] = t - m - lse


def kernel(x, edge_index, W1, b1, W2, b2):
    n, d_in = x.shape
    h_dim = W1.shape[1]
    c_dim = W2.shape[1]
    e = edge_index.shape[1]
    et = e + n

    cpt = _round_up(-(-et // (NW * LANE)), RING)
    epad = NW * cpt * LANE
    rpt = _round_up(-(-(n + 1) // NS), 16)
    npad = rpt * NS

    f32 = jnp.float32
    i32 = jnp.int32
    loops = jnp.arange(n, dtype=i32)
    pad_dst = n + jnp.arange(epad - et, dtype=i32) % (npad - n)
    src = jnp.concatenate([edge_index[0], loops,
                           jnp.zeros((epad - et,), i32)]).reshape(NW, cpt, LANE)
    dst = jnp.concatenate([edge_index[1], loops,
                           pad_dst]).reshape(NW, cpt, LANE)
    zeros1 = jnp.zeros((npad,), f32)
    zeros2 = jnp.zeros((npad, h_dim), f32)
    ones = jnp.ones((LANE,), f32)

    deg_parts = _make_deg_kernel(npad, cpt)(dst, zeros1, ones).reshape(NC, npad)
    agg1 = _make_agg_kernel(n, npad, cpt, h_dim)
    agg2 = _make_agg_kernel(n, npad, cpt, h_dim, fuse_stage=True)

    blk = n
    grid = (1,)

    hn1, dinv = pl.pallas_call(
        _tc1_body,
        grid=grid,
        in_specs=[
            pl.BlockSpec((blk, d_in), lambda i: (i, 0)),
            pl.BlockSpec((d_in, h_dim), lambda i: (0, 0)),
            pl.BlockSpec((blk, 1), lambda i: (i, 0)),
            pl.BlockSpec((blk, 1), lambda i: (i, 0)),
        ],
        out_specs=[
            pl.BlockSpec((blk, h_dim), lambda i: (i, 0)),
            pl.BlockSpec((blk, 1), lambda i: (i, 0)),
        ],
        out_shape=[
            jax.ShapeDtypeStruct((n, h_dim), f32),
            jax.ShapeDtypeStruct((n, 1), f32),
        ],
    )(x, W1, deg_parts[0, :n, None], deg_parts[1, :n, None])

    p = agg1(hn1, src, dst, zeros2)

    dinv_rep = jnp.concatenate(
        [jnp.broadcast_to(dinv, (n, h_dim)),
         jnp.ones((npad - n, h_dim), f32)], axis=0)
    q = agg2(p, dinv_rep, b1, src, dst, zeros2)

    out = pl.pallas_call(
        _tc3_body,
        grid=grid,
        in_specs=[
            pl.BlockSpec((blk, h_dim), lambda i: (i, 0)),
            pl.BlockSpec((blk, h_dim), lambda i: (i, 0)),
            pl.BlockSpec((h_dim, c_dim), lambda i: (0, 0)),
            pl.BlockSpec((1, c_dim), lambda i: (0, 0)),
            pl.BlockSpec((blk, 1), lambda i: (i, 0)),
        ],
        out_specs=pl.BlockSpec((blk, c_dim), lambda i: (i, 0)),
        out_shape=jax.ShapeDtypeStruct((n, c_dim), f32),
    )(q[0, :n], q[1, :n], W2, b2[None, :], dinv)

    return out
